# baseline (speedup 1.0000x reference)
import functools

import jax
import jax.numpy as jnp
from jax import lax
from jax.experimental import pallas as pl
from jax.experimental.pallas import tpu as pltpu
from jax.experimental.pallas import tpu_sc as plsc

HIDDEN = 1024
SHARED = 4096
PRIVATE = 1024
N_EXPERTS = 4096
N_HEADS = 4
K_PER_HEAD = 4
NUM_KEYS = 64
DHALF = PRIVATE // 2
T = 2048

TT = 256
NEG = float("-inf")


def _h_body(x_ref, wup_ref, wdn_ref, h_ref, wupb_ref):
    @pl.when(pl.program_id(0) == 0)
    def _cast():
        wupb_ref[...] = wup_ref[...].astype(jnp.bfloat16)

    xb = x_ref[...].astype(jnp.bfloat16)
    mid = jnp.dot(xb, wupb_ref[...], preferred_element_type=jnp.float32)
    midb = jax.nn.silu(mid).astype(jnp.bfloat16)
    h_ref[...] = jnp.dot(midb, wdn_ref[...], preferred_element_type=jnp.float32)


def _stage_h(x, wup, wdn_b):
    return pl.pallas_call(
        _h_body,
        grid=(T // TT,),
        in_specs=[
            pl.BlockSpec((TT, HIDDEN), lambda i: (i, 0)),
            pl.BlockSpec((HIDDEN, SHARED), lambda i: (0, 0)),
            pl.BlockSpec((SHARED, PRIVATE), lambda i: (0, 0)),
        ],
        out_specs=pl.BlockSpec((TT, PRIVATE), lambda i: (i, 0)),
        out_shape=jax.ShapeDtypeStruct((T, PRIVATE), jnp.float32),
        scratch_shapes=[pltpu.VMEM((HIDDEN, SHARED), jnp.bfloat16)],
    )(x, wup, wdn_b)


def _top4_t(s):
    n = s.shape[0]
    iota = lax.broadcasted_iota(jnp.int32, s.shape, 0)
    vals, poss = [], []
    for _ in range(K_PER_HEAD):
        m = jnp.max(s, axis=0, keepdims=True)
        hit = s == m
        pos = jnp.min(jnp.where(hit, iota, n), axis=0, keepdims=True)
        vals.append(m)
        poss.append(pos)
        s = jnp.where(iota == pos, NEG, s)
    return vals, poss


def _route_body(h_ref, wq_ref, kbdt_ref, up_ref, h2_ref, idx_ref, pw_ref,
                upb_ref):
    @pl.when(pl.program_id(0) == 0)
    def _cast():
        upb_ref[...] = up_ref[...].astype(jnp.bfloat16)

    hb = h_ref[...].astype(jnp.bfloat16)
    q = jnp.dot(hb, wq_ref[...], preferred_element_type=jnp.float32)
    h2_ref[...] = lax.dot_general(
        hb, upb_ref[...], (((1,), (1,)), ((), ())),
        preferred_element_type=jnp.float32)
    qb = q.astype(jnp.bfloat16)
    sim = jnp.dot(qb, kbdt_ref[...], preferred_element_type=jnp.float32)
    simt = sim.T

    idx_rows, pw_rows = [], []
    for hh in range(N_HEADS):
        rx = hh * NUM_KEYS
        ry = (N_HEADS + hh) * NUM_KEYS
        vx, ix = _top4_t(simt[rx:rx + NUM_KEYS, :])
        vy, iy = _top4_t(simt[ry:ry + NUM_KEYS, :])
        all_s = jnp.concatenate(
            [vx[i] + vy[j] for i in range(4) for j in range(4)], axis=0)
        all_i = jnp.concatenate(
            [ix[i] * NUM_KEYS + iy[j] for i in range(4) for j in range(4)], axis=0)
        iota16 = lax.broadcasted_iota(jnp.int32, all_s.shape, 0)
        s = all_s
        svals, eidx = [], []
        for _ in range(K_PER_HEAD):
            m = jnp.max(s, axis=0, keepdims=True)
            hit = s == m
            pos = jnp.min(jnp.where(hit, iota16, 16), axis=0, keepdims=True)
            e = jnp.sum(jnp.where(iota16 == pos, all_i, 0), axis=0, keepdims=True)
            svals.append(m)
            eidx.append(e)
            s = jnp.where(iota16 == pos, NEG, s)
        sc = jnp.concatenate(svals, axis=0)
        mx = jnp.max(sc, axis=0, keepdims=True)
        ex = jnp.exp(sc - mx)
        pw = ex / jnp.sum(ex, axis=0, keepdims=True)
        idx_rows.extend(eidx)
        pw_rows.append(pw)
    idx_ref[...] = jnp.concatenate(idx_rows, axis=0)
    pw_ref[...] = jnp.concatenate(pw_rows, axis=0)


def _stage_route(h, wq_b, kbdt_b, up):
    return pl.pallas_call(
        _route_body,
        grid=(T // TT,),
        in_specs=[
            pl.BlockSpec((TT, PRIVATE), lambda i: (i, 0)),
            pl.BlockSpec((PRIVATE, 2 * N_HEADS * DHALF), lambda i: (0, 0)),
            pl.BlockSpec((2 * N_HEADS * DHALF, 2 * N_HEADS * NUM_KEYS),
                         lambda i: (0, 0)),
            pl.BlockSpec((N_EXPERTS, PRIVATE), lambda i: (0, 0)),
        ],
        out_specs=[
            pl.BlockSpec((TT, N_EXPERTS), lambda i: (i, 0)),
            pl.BlockSpec((16, TT), lambda i: (0, i)),
            pl.BlockSpec((16, TT), lambda i: (0, i)),
        ],
        out_shape=[
            jax.ShapeDtypeStruct((T, N_EXPERTS), jnp.float32),
            jax.ShapeDtypeStruct((16, T), jnp.int32),
            jax.ShapeDtypeStruct((16, T), jnp.float32),
        ],
        scratch_shapes=[pltpu.VMEM((N_EXPERTS, PRIVATE), jnp.bfloat16)],
    )(h, wq_b, kbdt_b, up)


_NC, _NS = 2, 16
_NW = _NC * _NS
_TPW = T // _NW
_TBLK = 8


def _sc_combine(h2, idxt, pwt):
    mesh = plsc.VectorSubcoreMesh(core_axis_name="c", subcore_axis_name="s")

    @functools.partial(
        pl.kernel,
        mesh=mesh,
        out_type=jax.ShapeDtypeStruct((T, N_EXPERTS), jnp.float32),
        compiler_params=pltpu.CompilerParams(needs_layout_passes=False),
        scratch_types=[
            pltpu.VMEM((16, 128), jnp.int32),
            pltpu.VMEM((16, 128), jnp.float32),
            pltpu.VMEM((_TBLK, N_EXPERTS), jnp.float32),
            pltpu.VMEM((_TBLK, N_EXPERTS), jnp.float32),
        ],
    )
    def sck(h2_hbm, idx_hbm, pw_hbm, s_hbm, idx_v, pw_v, hbuf, sbuf):
        wid = lax.axis_index("s") * _NC + lax.axis_index("c")
        base = wid * _TPW
        slab = (wid // 2) * 128
        off = (wid % 2) * _TPW
        pltpu.sync_copy(idx_hbm.at[:, pl.ds(slab, 128)], idx_v)
        pltpu.sync_copy(pw_hbm.at[:, pl.ds(slab, 128)], pw_v)

        def zero_body(i, carry):
            r = i // (N_EXPERTS // 16)
            c = (i % (N_EXPERTS // 16)) * 16
            sbuf[r, pl.ds(c, 16)] = jnp.zeros((16,), jnp.float32)
            return carry

        lax.fori_loop(0, _TBLK * (N_EXPERTS // 16), zero_body, 0)

        def blk_body(bb, carry):
            t0 = base + bb * _TBLK
            pltpu.sync_copy(h2_hbm.at[pl.ds(t0, _TBLK)], hbuf)
            lane = lax.iota(jnp.int32, 16)
            for i in range(_TBLK):
                tl = bb * _TBLK + i
                rowi = jnp.full((16,), i, jnp.int32)
                e16 = plsc.load_gather(
                    idx_v, [lane, jnp.full((16,), off + tl, jnp.int32)])
                x16 = plsc.load_gather(hbuf, [rowi, e16])
                pw16 = plsc.load_gather(
                    pw_v, [lane, jnp.full((16,), off + tl, jnp.int32)])
                w = x16 * pw16 / (1.0 + jnp.exp(-x16))
                for hh in range(N_HEADS):
                    plsc.addupdate_scatter(
                        sbuf, [rowi, e16], w, mask=(lane // 4) == hh)
            pltpu.sync_copy(sbuf, s_hbm.at[pl.ds(t0, _TBLK)])
            for i in range(_TBLK):
                tl = bb * _TBLK + i
                rowi = jnp.full((16,), i, jnp.int32)
                e16 = plsc.load_gather(
                    idx_v, [lane, jnp.full((16,), off + tl, jnp.int32)])
                plsc.store_scatter(sbuf, [rowi, e16],
                                   jnp.zeros((16,), jnp.float32))
            return carry

        lax.fori_loop(0, _TPW // _TBLK, blk_body, 0)

    return sck(h2, idxt, pwt)


def _out_body(s_ref, down_ref, o_ref, dnb_ref):
    @pl.when(pl.program_id(0) == 0)
    def _cast():
        dnb_ref[...] = down_ref[...].astype(jnp.bfloat16)

    sb = s_ref[...].astype(jnp.bfloat16)
    o_ref[...] = jnp.dot(sb, dnb_ref[...], preferred_element_type=jnp.float32)


def _stage_out(s, down):
    return pl.pallas_call(
        _out_body,
        grid=(T // TT,),
        in_specs=[
            pl.BlockSpec((TT, N_EXPERTS), lambda i: (i, 0)),
            pl.BlockSpec((N_EXPERTS, HIDDEN), lambda i: (0, 0)),
        ],
        out_specs=pl.BlockSpec((TT, HIDDEN), lambda i: (i, 0)),
        out_shape=jax.ShapeDtypeStruct((T, HIDDEN), jnp.float32),
        scratch_shapes=[pltpu.VMEM((N_EXPERTS, HIDDEN), jnp.bfloat16)],
    )(s, down)


def kernel(hidden_states, W_up, W_down, W_q, keys, up_embed, down_embed):
    x = hidden_states.reshape(T, HIDDEN)
    wdn_b = W_down.astype(jnp.bfloat16)
    wq_b = W_q.astype(jnp.bfloat16)
    kk = keys.astype(jnp.bfloat16).transpose(2, 0, 1, 3).reshape(
        8, NUM_KEYS, DHALF)
    kbdt_b = jax.scipy.linalg.block_diag(*[kk[i].T for i in range(8)])

    h = _stage_h(x, W_up, wdn_b)
    h2, idxt, pwt = _stage_route(h, wq_b, kbdt_b, up_embed)
    s = _sc_combine(h2, idxt, pwt)
    out = _stage_out(s, down_embed)
    return out.reshape(1, T, HIDDEN)


# speedup vs baseline: 1.0203x; 1.0203x over previous
import functools

import jax
import jax.numpy as jnp
from jax import lax
from jax.experimental import pallas as pl
from jax.experimental.pallas import tpu as pltpu
from jax.experimental.pallas import tpu_sc as plsc

HIDDEN = 1024
SHARED = 4096
PRIVATE = 1024
N_EXPERTS = 4096
N_HEADS = 4
K_PER_HEAD = 4
NUM_KEYS = 64
DHALF = PRIVATE // 2
T = 2048

TT = 256
NEG = float("-inf")


def _h_body(x_ref, wup_ref, wdn_ref, h_ref, wupb_ref):
    @pl.when(pl.program_id(0) == 0)
    def _cast():
        wupb_ref[...] = wup_ref[...].astype(jnp.bfloat16)

    xb = x_ref[...].astype(jnp.bfloat16)
    mid = jnp.dot(xb, wupb_ref[...], preferred_element_type=jnp.float32)
    midb = jax.nn.silu(mid).astype(jnp.bfloat16)
    h_ref[...] = jnp.dot(midb, wdn_ref[...], preferred_element_type=jnp.float32)


def _stage_h(x, wup, wdn_b):
    return pl.pallas_call(
        _h_body,
        grid=(T // TT,),
        in_specs=[
            pl.BlockSpec((TT, HIDDEN), lambda i: (i, 0)),
            pl.BlockSpec((HIDDEN, SHARED), lambda i: (0, 0)),
            pl.BlockSpec((SHARED, PRIVATE), lambda i: (0, 0)),
        ],
        out_specs=pl.BlockSpec((TT, PRIVATE), lambda i: (i, 0)),
        out_shape=jax.ShapeDtypeStruct((T, PRIVATE), jnp.float32),
        scratch_shapes=[pltpu.VMEM((HIDDEN, SHARED), jnp.bfloat16)],
    )(x, wup, wdn_b)


def _top4_t(s):
    n = s.shape[0]
    iota = lax.broadcasted_iota(jnp.int32, s.shape, 0)
    vals, poss = [], []
    for _ in range(K_PER_HEAD):
        m = jnp.max(s, axis=0, keepdims=True)
        hit = s == m
        pos = jnp.min(jnp.where(hit, iota, n), axis=0, keepdims=True)
        vals.append(m)
        poss.append(pos)
        s = jnp.where(iota == pos, NEG, s)
    return vals, poss


def _route_body(h_ref, wq_ref, kbdt_ref, up_ref, h2_ref, idx_ref, pw_ref,
                upb_ref):
    @pl.when(pl.program_id(0) == 0)
    def _cast():
        upb_ref[...] = up_ref[...].astype(jnp.bfloat16)

    hb = h_ref[...].astype(jnp.bfloat16)
    q = jnp.dot(hb, wq_ref[...], preferred_element_type=jnp.float32)
    h2_ref[...] = lax.dot_general(
        hb, upb_ref[...], (((1,), (1,)), ((), ())),
        preferred_element_type=jnp.float32)
    qb = q.astype(jnp.bfloat16)
    sim = jnp.dot(qb, kbdt_ref[...], preferred_element_type=jnp.float32)
    simt = sim.T

    idx_rows, pw_rows = [], []
    for hh in range(N_HEADS):
        rx = hh * NUM_KEYS
        ry = (N_HEADS + hh) * NUM_KEYS
        vx, ix = _top4_t(simt[rx:rx + NUM_KEYS, :])
        vy, iy = _top4_t(simt[ry:ry + NUM_KEYS, :])
        all_s = jnp.concatenate(
            [vx[i] + vy[j] for i in range(4) for j in range(4)], axis=0)
        all_i = jnp.concatenate(
            [ix[i] * NUM_KEYS + iy[j] for i in range(4) for j in range(4)], axis=0)
        iota16 = lax.broadcasted_iota(jnp.int32, all_s.shape, 0)
        s = all_s
        svals, eidx = [], []
        for _ in range(K_PER_HEAD):
            m = jnp.max(s, axis=0, keepdims=True)
            hit = s == m
            pos = jnp.min(jnp.where(hit, iota16, 16), axis=0, keepdims=True)
            e = jnp.sum(jnp.where(iota16 == pos, all_i, 0), axis=0, keepdims=True)
            svals.append(m)
            eidx.append(e)
            s = jnp.where(iota16 == pos, NEG, s)
        sc = jnp.concatenate(svals, axis=0)
        mx = jnp.max(sc, axis=0, keepdims=True)
        ex = jnp.exp(sc - mx)
        pw = ex / jnp.sum(ex, axis=0, keepdims=True)
        idx_rows.extend(eidx)
        pw_rows.append(pw)
    idx_ref[...] = jnp.concatenate(idx_rows, axis=0)
    pw_ref[...] = jnp.concatenate(pw_rows, axis=0)


def _stage_route(h, wq_b, kbdt_b, up):
    return pl.pallas_call(
        _route_body,
        grid=(T // TT,),
        in_specs=[
            pl.BlockSpec((TT, PRIVATE), lambda i: (i, 0)),
            pl.BlockSpec((PRIVATE, 2 * N_HEADS * DHALF), lambda i: (0, 0)),
            pl.BlockSpec((2 * N_HEADS * DHALF, 2 * N_HEADS * NUM_KEYS),
                         lambda i: (0, 0)),
            pl.BlockSpec((N_EXPERTS, PRIVATE), lambda i: (0, 0)),
        ],
        out_specs=[
            pl.BlockSpec((TT, N_EXPERTS), lambda i: (i, 0)),
            pl.BlockSpec((16, TT), lambda i: (0, i)),
            pl.BlockSpec((16, TT), lambda i: (0, i)),
        ],
        out_shape=[
            jax.ShapeDtypeStruct((T, N_EXPERTS), jnp.float32),
            jax.ShapeDtypeStruct((16, T), jnp.int32),
            jax.ShapeDtypeStruct((16, T), jnp.float32),
        ],
        scratch_shapes=[pltpu.VMEM((N_EXPERTS, PRIVATE), jnp.bfloat16)],
    )(h, wq_b, kbdt_b, up)


_NC, _NS = 2, 16
_NW = _NC * _NS
_TPW = T // _NW
_TBLK = 4
_NBLK = _TPW // _TBLK


def _sc_combine(h2, idxt, pwt):
    mesh = plsc.VectorSubcoreMesh(core_axis_name="c", subcore_axis_name="s")

    @functools.partial(
        pl.kernel,
        mesh=mesh,
        out_type=jax.ShapeDtypeStruct((T, N_EXPERTS), jnp.float32),
        compiler_params=pltpu.CompilerParams(needs_layout_passes=False),
        scratch_types=[
            pltpu.VMEM((16, 128), jnp.int32),
            pltpu.VMEM((16, 128), jnp.float32),
            pltpu.VMEM((2, _TBLK, N_EXPERTS), jnp.float32),
            pltpu.VMEM((2, _TBLK, N_EXPERTS), jnp.float32),
            pltpu.SemaphoreType.DMA,
            pltpu.SemaphoreType.DMA,
            pltpu.SemaphoreType.DMA,
            pltpu.SemaphoreType.DMA,
        ],
    )
    def sck(h2_hbm, idx_hbm, pw_hbm, s_hbm, idx_v, pw_v, hbuf, sbuf,
            isem0, isem1, osem0, osem1):
        wid = lax.axis_index("s") * _NC + lax.axis_index("c")
        base = wid * _TPW
        slab = (wid // 2) * 128
        off = (wid % 2) * _TPW
        isems = [isem0, isem1]
        osems = [osem0, osem1]
        pltpu.sync_copy(idx_hbm.at[:, pl.ds(slab, 128)], idx_v)
        pltpu.sync_copy(pw_hbm.at[:, pl.ds(slab, 128)], pw_v)

        def zero_body(i, carry):
            k = i // (_TBLK * (N_EXPERTS // 16))
            j = i % (_TBLK * (N_EXPERTS // 16))
            r = j // (N_EXPERTS // 16)
            c = (j % (N_EXPERTS // 16)) * 16
            sbuf[k, r, pl.ds(c, 16)] = jnp.zeros((16,), jnp.float32)
            return carry

        lax.fori_loop(0, 2 * _TBLK * (N_EXPERTS // 16), zero_body, 0)

        def in_copy(bb, k):
            return pltpu.make_async_copy(
                h2_hbm.at[pl.ds(base + bb * _TBLK, _TBLK)], hbuf.at[k],
                isems[k])

        def out_copy(bb, k):
            return pltpu.make_async_copy(
                sbuf.at[k], s_hbm.at[pl.ds(base + bb * _TBLK, _TBLK)],
                osems[k])

        lane = lax.iota(jnp.int32, 16)
        in_copy(0, 0).start()
        for bb in range(_NBLK):
            k = bb % 2
            if bb + 1 < _NBLK:
                in_copy(bb + 1, 1 - k).start()
            in_copy(bb, k).wait()
            if bb >= 2:
                out_copy(bb - 2, k).wait()
                for i in range(_TBLK):
                    tl = (bb - 2) * _TBLK + i
                    rowi = jnp.full((16,), i, jnp.int32)
                    e16 = plsc.load_gather(
                        idx_v, [lane, jnp.full((16,), off + tl, jnp.int32)])
                    plsc.store_scatter(sbuf, [jnp.full((16,), k, jnp.int32),
                                              rowi, e16],
                                       jnp.zeros((16,), jnp.float32))
            for i in range(_TBLK):
                tl = bb * _TBLK + i
                rowi = jnp.full((16,), i, jnp.int32)
                k16 = jnp.full((16,), k, jnp.int32)
                e16 = plsc.load_gather(
                    idx_v, [lane, jnp.full((16,), off + tl, jnp.int32)])
                x16 = plsc.load_gather(hbuf, [k16, rowi, e16])
                pw16 = plsc.load_gather(
                    pw_v, [lane, jnp.full((16,), off + tl, jnp.int32)])
                w = x16 * pw16 / (1.0 + jnp.exp(-x16))
                for hh in range(N_HEADS):
                    plsc.addupdate_scatter(
                        sbuf, [k16, rowi, e16], w, mask=(lane // 4) == hh)
            out_copy(bb, k).start()
        out_copy(_NBLK - 2, _NBLK % 2).wait()
        out_copy(_NBLK - 1, (_NBLK - 1) % 2).wait()

    return sck(h2, idxt, pwt)


def _out_body(s_ref, down_ref, o_ref, dnb_ref):
    @pl.when(pl.program_id(0) == 0)
    def _cast():
        dnb_ref[...] = down_ref[...].astype(jnp.bfloat16)

    sb = s_ref[...].astype(jnp.bfloat16)
    o_ref[...] = jnp.dot(sb, dnb_ref[...], preferred_element_type=jnp.float32)


def _stage_out(s, down):
    return pl.pallas_call(
        _out_body,
        grid=(T // TT,),
        in_specs=[
            pl.BlockSpec((TT, N_EXPERTS), lambda i: (i, 0)),
            pl.BlockSpec((N_EXPERTS, HIDDEN), lambda i: (0, 0)),
        ],
        out_specs=pl.BlockSpec((TT, HIDDEN), lambda i: (i, 0)),
        out_shape=jax.ShapeDtypeStruct((T, HIDDEN), jnp.float32),
        scratch_shapes=[pltpu.VMEM((N_EXPERTS, HIDDEN), jnp.bfloat16)],
    )(s, down)


def kernel(hidden_states, W_up, W_down, W_q, keys, up_embed, down_embed):
    x = hidden_states.reshape(T, HIDDEN)
    wdn_b = W_down.astype(jnp.bfloat16)
    wq_b = W_q.astype(jnp.bfloat16)
    kk = keys.astype(jnp.bfloat16).transpose(2, 0, 1, 3).reshape(
        8, NUM_KEYS, DHALF)
    kbdt_b = jax.scipy.linalg.block_diag(*[kk[i].T for i in range(8)])

    h = _stage_h(x, W_up, wdn_b)
    h2, idxt, pwt = _stage_route(h, wq_b, kbdt_b, up_embed)
    s = _sc_combine(h2, idxt, pwt)
    out = _stage_out(s, down_embed)
    return out.reshape(1, T, HIDDEN)


# speedup vs baseline: 1.0502x; 1.0293x over previous
import functools

import jax
import jax.numpy as jnp
from jax import lax
from jax.experimental import pallas as pl
from jax.experimental.pallas import tpu as pltpu
from jax.experimental.pallas import tpu_sc as plsc

HIDDEN = 1024
SHARED = 4096
PRIVATE = 1024
N_EXPERTS = 4096
N_HEADS = 4
K_PER_HEAD = 4
NUM_KEYS = 64
DHALF = PRIVATE // 2
T = 2048

TT = 256
NEG = float("-inf")


def _h_body(x_ref, wup_ref, wdn_ref, h_ref, wupb_ref):
    @pl.when(pl.program_id(0) == 0)
    def _cast():
        wupb_ref[...] = wup_ref[...].astype(jnp.bfloat16)

    xb = x_ref[...].astype(jnp.bfloat16)
    mid = jnp.dot(xb, wupb_ref[...], preferred_element_type=jnp.float32)
    midb = jax.nn.silu(mid).astype(jnp.bfloat16)
    h_ref[...] = jnp.dot(midb, wdn_ref[...], preferred_element_type=jnp.float32)


def _stage_h(x, wup, wdn_b):
    return pl.pallas_call(
        _h_body,
        grid=(T // TT,),
        in_specs=[
            pl.BlockSpec((TT, HIDDEN), lambda i: (i, 0)),
            pl.BlockSpec((HIDDEN, SHARED), lambda i: (0, 0)),
            pl.BlockSpec((SHARED, PRIVATE), lambda i: (0, 0)),
        ],
        out_specs=pl.BlockSpec((TT, PRIVATE), lambda i: (i, 0)),
        out_shape=jax.ShapeDtypeStruct((T, PRIVATE), jnp.float32),
        scratch_shapes=[pltpu.VMEM((HIDDEN, SHARED), jnp.bfloat16)],
    )(x, wup, wdn_b)


def _top4_t(s):
    n = s.shape[0]
    iota = lax.broadcasted_iota(jnp.int32, s.shape, 0)
    vals, poss = [], []
    for _ in range(K_PER_HEAD):
        m = jnp.max(s, axis=0, keepdims=True)
        hit = s == m
        pos = jnp.min(jnp.where(hit, iota, n), axis=0, keepdims=True)
        vals.append(m)
        poss.append(pos)
        s = jnp.where(iota == pos, NEG, s)
    return vals, poss


def _route_body(h_ref, wq_ref, kbdt_ref, up_ref, h2_ref, idx_ref, pw_ref,
                upb_ref):
    @pl.when(pl.program_id(0) == 0)
    def _cast():
        upb_ref[...] = up_ref[...].astype(jnp.bfloat16)

    hb = h_ref[...].astype(jnp.bfloat16)
    q = jnp.dot(hb, wq_ref[...], preferred_element_type=jnp.float32)
    h2_ref[...] = lax.dot_general(
        hb, upb_ref[...], (((1,), (1,)), ((), ())),
        preferred_element_type=jnp.float32)
    qb = q.astype(jnp.bfloat16)
    sim = jnp.dot(qb, kbdt_ref[...], preferred_element_type=jnp.float32)
    simt = sim.T

    idx_rows, pw_rows = [], []
    for hh in range(N_HEADS):
        rx = hh * NUM_KEYS
        ry = (N_HEADS + hh) * NUM_KEYS
        vx, ix = _top4_t(simt[rx:rx + NUM_KEYS, :])
        vy, iy = _top4_t(simt[ry:ry + NUM_KEYS, :])
        all_s = jnp.concatenate(
            [vx[i] + vy[j] for i in range(4) for j in range(4)], axis=0)
        all_i = jnp.concatenate(
            [ix[i] * NUM_KEYS + iy[j] for i in range(4) for j in range(4)], axis=0)
        iota16 = lax.broadcasted_iota(jnp.int32, all_s.shape, 0)
        s = all_s
        svals, eidx = [], []
        for _ in range(K_PER_HEAD):
            m = jnp.max(s, axis=0, keepdims=True)
            hit = s == m
            pos = jnp.min(jnp.where(hit, iota16, 16), axis=0, keepdims=True)
            e = jnp.sum(jnp.where(iota16 == pos, all_i, 0), axis=0, keepdims=True)
            svals.append(m)
            eidx.append(e)
            s = jnp.where(iota16 == pos, NEG, s)
        sc = jnp.concatenate(svals, axis=0)
        mx = jnp.max(sc, axis=0, keepdims=True)
        ex = jnp.exp(sc - mx)
        pw = ex / jnp.sum(ex, axis=0, keepdims=True)
        idx_rows.extend(eidx)
        pw_rows.append(pw)
    idx_ref[...] = jnp.concatenate(idx_rows, axis=0)
    pw_ref[...] = jnp.concatenate(pw_rows, axis=0)


def _stage_route(h, wq_b, kbdt_b, up):
    return pl.pallas_call(
        _route_body,
        grid=(T // TT,),
        in_specs=[
            pl.BlockSpec((TT, PRIVATE), lambda i: (i, 0)),
            pl.BlockSpec((PRIVATE, 2 * N_HEADS * DHALF), lambda i: (0, 0)),
            pl.BlockSpec((2 * N_HEADS * DHALF, 2 * N_HEADS * NUM_KEYS),
                         lambda i: (0, 0)),
            pl.BlockSpec((N_EXPERTS, PRIVATE), lambda i: (0, 0)),
        ],
        out_specs=[
            pl.BlockSpec((TT, N_EXPERTS), lambda i: (i, 0)),
            pl.BlockSpec((16, TT), lambda i: (0, i)),
            pl.BlockSpec((16, TT), lambda i: (0, i)),
        ],
        out_shape=[
            jax.ShapeDtypeStruct((T, N_EXPERTS), jnp.float32),
            jax.ShapeDtypeStruct((16, T), jnp.int32),
            jax.ShapeDtypeStruct((16, T), jnp.float32),
        ],
        scratch_shapes=[pltpu.VMEM((N_EXPERTS, PRIVATE), jnp.bfloat16)],
    )(h, wq_b, kbdt_b, up)


_NC, _NS = 2, 16
_NW = _NC * _NS
_TPW = T // _NW
_TBLK = 4
_NBLK = _TPW // _TBLK


def _sc_combine(h2, idxt, pwt):
    mesh = plsc.VectorSubcoreMesh(core_axis_name="c", subcore_axis_name="s")

    @functools.partial(
        pl.kernel,
        mesh=mesh,
        out_type=jax.ShapeDtypeStruct((T, N_EXPERTS), jnp.float32),
        compiler_params=pltpu.CompilerParams(needs_layout_passes=False),
        scratch_types=[
            pltpu.VMEM((16, 128), jnp.int32),
            pltpu.VMEM((16, 128), jnp.float32),
            pltpu.VMEM((2, _TBLK, N_EXPERTS), jnp.float32),
            pltpu.VMEM((2, _TBLK, N_EXPERTS), jnp.float32),
            pltpu.SemaphoreType.DMA,
            pltpu.SemaphoreType.DMA,
            pltpu.SemaphoreType.DMA,
            pltpu.SemaphoreType.DMA,
        ],
    )
    def sck(h2_hbm, idx_hbm, pw_hbm, s_hbm, idx_v, pw_v, hbuf, sbuf,
            isem0, isem1, osem0, osem1):
        wid = lax.axis_index("s") * _NC + lax.axis_index("c")
        base = wid * _TPW
        slab = (wid // 2) * 128
        off = (wid % 2) * _TPW
        isems = [isem0, isem1]
        osems = [osem0, osem1]
        pltpu.sync_copy(idx_hbm.at[:, pl.ds(slab, 128)], idx_v)
        pltpu.sync_copy(pw_hbm.at[:, pl.ds(slab, 128)], pw_v)

        for k0 in range(2):
            for r0 in range(_TBLK):
                def zero_body(ci, carry, k0=k0, r0=r0):
                    for j in range(16):
                        sbuf[k0, r0, pl.ds(ci * 256 + j * 16, 16)] = (
                            jnp.zeros((16,), jnp.float32))
                    return carry

                lax.fori_loop(0, N_EXPERTS // 256, zero_body, 0)

        def in_copy(bb, k):
            return pltpu.make_async_copy(
                h2_hbm.at[pl.ds(base + bb * _TBLK, _TBLK)], hbuf.at[k],
                isems[k])

        def out_copy(bb, k):
            return pltpu.make_async_copy(
                sbuf.at[k], s_hbm.at[pl.ds(base + bb * _TBLK, _TBLK)],
                osems[k])

        lane = lax.iota(jnp.int32, 16)
        in_copy(0, 0).start()
        for bb in range(_NBLK):
            k = bb % 2
            if bb + 1 < _NBLK:
                in_copy(bb + 1, 1 - k).start()
            in_copy(bb, k).wait()
            if bb >= 2:
                out_copy(bb - 2, k).wait()
                for i in range(_TBLK):
                    tl = (bb - 2) * _TBLK + i
                    rowi = jnp.full((16,), i, jnp.int32)
                    e16 = plsc.load_gather(
                        idx_v, [lane, jnp.full((16,), off + tl, jnp.int32)])
                    plsc.store_scatter(sbuf, [jnp.full((16,), k, jnp.int32),
                                              rowi, e16],
                                       jnp.zeros((16,), jnp.float32))
            for i in range(_TBLK):
                tl = bb * _TBLK + i
                rowi = jnp.full((16,), i, jnp.int32)
                k16 = jnp.full((16,), k, jnp.int32)
                e16 = plsc.load_gather(
                    idx_v, [lane, jnp.full((16,), off + tl, jnp.int32)])
                x16 = plsc.load_gather(hbuf, [k16, rowi, e16])
                pw16 = plsc.load_gather(
                    pw_v, [lane, jnp.full((16,), off + tl, jnp.int32)])
                w = x16 * pw16 / (1.0 + jnp.exp(-x16))
                for hh in range(N_HEADS):
                    plsc.addupdate_scatter(
                        sbuf, [k16, rowi, e16], w, mask=(lane // 4) == hh)
            out_copy(bb, k).start()
        out_copy(_NBLK - 2, _NBLK % 2).wait()
        out_copy(_NBLK - 1, (_NBLK - 1) % 2).wait()

    return sck(h2, idxt, pwt)


def _out_body(s_ref, down_ref, o_ref, dnb_ref):
    @pl.when(pl.program_id(0) == 0)
    def _cast():
        dnb_ref[...] = down_ref[...].astype(jnp.bfloat16)

    sb = s_ref[...].astype(jnp.bfloat16)
    o_ref[...] = jnp.dot(sb, dnb_ref[...], preferred_element_type=jnp.float32)


def _stage_out(s, down):
    return pl.pallas_call(
        _out_body,
        grid=(T // TT,),
        in_specs=[
            pl.BlockSpec((TT, N_EXPERTS), lambda i: (i, 0)),
            pl.BlockSpec((N_EXPERTS, HIDDEN), lambda i: (0, 0)),
        ],
        out_specs=pl.BlockSpec((TT, HIDDEN), lambda i: (i, 0)),
        out_shape=jax.ShapeDtypeStruct((T, HIDDEN), jnp.float32),
        scratch_shapes=[pltpu.VMEM((N_EXPERTS, HIDDEN), jnp.bfloat16)],
    )(s, down)


def kernel(hidden_states, W_up, W_down, W_q, keys, up_embed, down_embed):
    x = hidden_states.reshape(T, HIDDEN)
    wdn_b = W_down.astype(jnp.bfloat16)
    wq_b = W_q.astype(jnp.bfloat16)
    kk = keys.astype(jnp.bfloat16).transpose(2, 0, 1, 3).reshape(
        8, NUM_KEYS, DHALF)
    kbdt_b = jax.scipy.linalg.block_diag(*[kk[i].T for i in range(8)])

    h = _stage_h(x, W_up, wdn_b)
    h2, idxt, pwt = _stage_route(h, wq_b, kbdt_b, up_embed)
    s = _sc_combine(h2, idxt, pwt)
    out = _stage_out(s, down_embed)
    return out.reshape(1, T, HIDDEN)


# speedup vs baseline: 1.1017x; 1.0491x over previous
import functools

import jax
import jax.numpy as jnp
from jax import lax
from jax.experimental import pallas as pl
from jax.experimental.pallas import tpu as pltpu
from jax.experimental.pallas import tpu_sc as plsc

HIDDEN = 1024
SHARED = 4096
PRIVATE = 1024
N_EXPERTS = 4096
N_HEADS = 4
K_PER_HEAD = 4
NUM_KEYS = 64
DHALF = PRIVATE // 2
T = 2048

TT = 256
NEG = float("-inf")


def _h_body(x_ref, wup_ref, wdn_ref, h_ref, wupb_ref, wdnb_ref):
    @pl.when(pl.program_id(0) == 0)
    def _cast():
        wupb_ref[...] = wup_ref[...].astype(jnp.bfloat16)
        wdnb_ref[...] = wdn_ref[...].astype(jnp.bfloat16)

    xb = x_ref[...].astype(jnp.bfloat16)
    mid = jnp.dot(xb, wupb_ref[...], preferred_element_type=jnp.float32)
    midb = jax.nn.silu(mid).astype(jnp.bfloat16)
    h_ref[...] = jnp.dot(midb, wdnb_ref[...], preferred_element_type=jnp.float32)


def _stage_h(x, wup, wdn):
    return pl.pallas_call(
        _h_body,
        grid=(T // TT,),
        in_specs=[
            pl.BlockSpec((TT, HIDDEN), lambda i: (i, 0)),
            pl.BlockSpec((HIDDEN, SHARED), lambda i: (0, 0)),
            pl.BlockSpec((SHARED, PRIVATE), lambda i: (0, 0)),
        ],
        out_specs=pl.BlockSpec((TT, PRIVATE), lambda i: (i, 0)),
        out_shape=jax.ShapeDtypeStruct((T, PRIVATE), jnp.float32),
        scratch_shapes=[pltpu.VMEM((HIDDEN, SHARED), jnp.bfloat16),
                        pltpu.VMEM((SHARED, PRIVATE), jnp.bfloat16)],
    )(x, wup, wdn)


def _top4_t(s):
    n = s.shape[0]
    iota = lax.broadcasted_iota(jnp.int32, s.shape, 0)
    vals, poss = [], []
    for _ in range(K_PER_HEAD):
        m = jnp.max(s, axis=0, keepdims=True)
        hit = s == m
        pos = jnp.min(jnp.where(hit, iota, n), axis=0, keepdims=True)
        vals.append(m)
        poss.append(pos)
        s = jnp.where(iota == pos, NEG, s)
    return vals, poss


def _route_body(h_ref, wq_ref, kbdt_ref, up_ref, h2_ref, idx_ref, pw_ref,
                upb_ref):
    @pl.when(pl.program_id(0) == 0)
    def _cast():
        upb_ref[...] = up_ref[...].astype(jnp.bfloat16)

    hb = h_ref[...].astype(jnp.bfloat16)
    q = jnp.dot(hb, wq_ref[...], preferred_element_type=jnp.float32)
    h2_ref[...] = lax.dot_general(
        hb, upb_ref[...], (((1,), (1,)), ((), ())),
        preferred_element_type=jnp.float32)
    qb = q.astype(jnp.bfloat16)
    sim = jnp.dot(qb, kbdt_ref[...], preferred_element_type=jnp.float32)
    simt = sim.T

    idx_rows, pw_rows = [], []
    for hh in range(N_HEADS):
        rx = hh * NUM_KEYS
        ry = (N_HEADS + hh) * NUM_KEYS
        vx, ix = _top4_t(simt[rx:rx + NUM_KEYS, :])
        vy, iy = _top4_t(simt[ry:ry + NUM_KEYS, :])
        all_s = jnp.concatenate(
            [vx[i] + vy[j] for i in range(4) for j in range(4)], axis=0)
        all_i = jnp.concatenate(
            [ix[i] * NUM_KEYS + iy[j] for i in range(4) for j in range(4)], axis=0)
        iota16 = lax.broadcasted_iota(jnp.int32, all_s.shape, 0)
        s = all_s
        svals, eidx = [], []
        for _ in range(K_PER_HEAD):
            m = jnp.max(s, axis=0, keepdims=True)
            hit = s == m
            pos = jnp.min(jnp.where(hit, iota16, 16), axis=0, keepdims=True)
            e = jnp.sum(jnp.where(iota16 == pos, all_i, 0), axis=0, keepdims=True)
            svals.append(m)
            eidx.append(e)
            s = jnp.where(iota16 == pos, NEG, s)
        sc = jnp.concatenate(svals, axis=0)
        mx = jnp.max(sc, axis=0, keepdims=True)
        ex = jnp.exp(sc - mx)
        pw = ex / jnp.sum(ex, axis=0, keepdims=True)
        idx_rows.extend(eidx)
        pw_rows.append(pw)
    idx_ref[...] = jnp.concatenate(idx_rows, axis=0)
    pw_ref[...] = jnp.concatenate(pw_rows, axis=0)


def _stage_route(h, wq_b, kbdt_b, up):
    return pl.pallas_call(
        _route_body,
        grid=(T // TT,),
        in_specs=[
            pl.BlockSpec((TT, PRIVATE), lambda i: (i, 0)),
            pl.BlockSpec((PRIVATE, 2 * N_HEADS * DHALF), lambda i: (0, 0)),
            pl.BlockSpec((2 * N_HEADS * DHALF, 2 * N_HEADS * NUM_KEYS),
                         lambda i: (0, 0)),
            pl.BlockSpec((N_EXPERTS, PRIVATE), lambda i: (0, 0)),
        ],
        out_specs=[
            pl.BlockSpec((TT, N_EXPERTS), lambda i: (i, 0)),
            pl.BlockSpec((16, TT), lambda i: (0, i)),
            pl.BlockSpec((16, TT), lambda i: (0, i)),
        ],
        out_shape=[
            jax.ShapeDtypeStruct((T, N_EXPERTS), jnp.float32),
            jax.ShapeDtypeStruct((16, T), jnp.int32),
            jax.ShapeDtypeStruct((16, T), jnp.float32),
        ],
        scratch_shapes=[pltpu.VMEM((N_EXPERTS, PRIVATE), jnp.bfloat16)],
    )(h, wq_b, kbdt_b, up)


_NC, _NS = 2, 16
_NW = _NC * _NS
_TPW = T // _NW
_TBLK = 4
_NBLK = _TPW // _TBLK


def _sc_combine(h2, idxt, pwt):
    mesh = plsc.VectorSubcoreMesh(core_axis_name="c", subcore_axis_name="s")

    @functools.partial(
        pl.kernel,
        mesh=mesh,
        out_type=jax.ShapeDtypeStruct((T, N_EXPERTS), jnp.float32),
        compiler_params=pltpu.CompilerParams(needs_layout_passes=False),
        scratch_types=[
            pltpu.VMEM((16, 128), jnp.int32),
            pltpu.VMEM((16, 128), jnp.float32),
            pltpu.VMEM((2, _TBLK, N_EXPERTS), jnp.float32),
            pltpu.VMEM((2, _TBLK, N_EXPERTS), jnp.float32),
            pltpu.SemaphoreType.DMA,
            pltpu.SemaphoreType.DMA,
            pltpu.SemaphoreType.DMA,
            pltpu.SemaphoreType.DMA,
        ],
    )
    def sck(h2_hbm, idx_hbm, pw_hbm, s_hbm, idx_v, pw_v, hbuf, sbuf,
            isem0, isem1, osem0, osem1):
        wid = lax.axis_index("s") * _NC + lax.axis_index("c")
        base = wid * _TPW
        slab = (wid // 2) * 128
        off = (wid % 2) * _TPW
        isems = [isem0, isem1]
        osems = [osem0, osem1]
        pltpu.sync_copy(idx_hbm.at[:, pl.ds(slab, 128)], idx_v)
        pltpu.sync_copy(pw_hbm.at[:, pl.ds(slab, 128)], pw_v)

        for k0 in range(2):
            for r0 in range(_TBLK):
                def zero_body(ci, carry, k0=k0, r0=r0):
                    for j in range(16):
                        sbuf[k0, r0, pl.ds(ci * 256 + j * 16, 16)] = (
                            jnp.zeros((16,), jnp.float32))
                    return carry

                lax.fori_loop(0, N_EXPERTS // 256, zero_body, 0)

        def in_copy(bb, k):
            return pltpu.make_async_copy(
                h2_hbm.at[pl.ds(base + bb * _TBLK, _TBLK)], hbuf.at[k],
                isems[k])

        def out_copy(bb, k):
            return pltpu.make_async_copy(
                sbuf.at[k], s_hbm.at[pl.ds(base + bb * _TBLK, _TBLK)],
                osems[k])

        lane = lax.iota(jnp.int32, 16)
        in_copy(0, 0).start()
        for bb in range(_NBLK):
            k = bb % 2
            if bb + 1 < _NBLK:
                in_copy(bb + 1, 1 - k).start()
            in_copy(bb, k).wait()
            if bb >= 2:
                out_copy(bb - 2, k).wait()
                for i in range(_TBLK):
                    tl = (bb - 2) * _TBLK + i
                    rowi = jnp.full((16,), i, jnp.int32)
                    e16 = plsc.load_gather(
                        idx_v, [lane, jnp.full((16,), off + tl, jnp.int32)])
                    plsc.store_scatter(sbuf, [jnp.full((16,), k, jnp.int32),
                                              rowi, e16],
                                       jnp.zeros((16,), jnp.float32))
            for i in range(_TBLK):
                tl = bb * _TBLK + i
                rowi = jnp.full((16,), i, jnp.int32)
                k16 = jnp.full((16,), k, jnp.int32)
                e16 = plsc.load_gather(
                    idx_v, [lane, jnp.full((16,), off + tl, jnp.int32)])
                x16 = plsc.load_gather(hbuf, [k16, rowi, e16])
                pw16 = plsc.load_gather(
                    pw_v, [lane, jnp.full((16,), off + tl, jnp.int32)])
                w = x16 * pw16 / (1.0 + jnp.exp(-x16))
                for hh in range(N_HEADS):
                    plsc.addupdate_scatter(
                        sbuf, [k16, rowi, e16], w, mask=(lane // 4) == hh)
            out_copy(bb, k).start()
        out_copy(_NBLK - 2, _NBLK % 2).wait()
        out_copy(_NBLK - 1, (_NBLK - 1) % 2).wait()

    return sck(h2, idxt, pwt)


def _out_body(s_ref, down_ref, o_ref, dnb_ref):
    @pl.when(pl.program_id(0) == 0)
    def _cast():
        dnb_ref[...] = down_ref[...].astype(jnp.bfloat16)

    sb = s_ref[...].astype(jnp.bfloat16)
    o_ref[...] = jnp.dot(sb, dnb_ref[...], preferred_element_type=jnp.float32)


def _stage_out(s, down):
    return pl.pallas_call(
        _out_body,
        grid=(T // TT,),
        in_specs=[
            pl.BlockSpec((TT, N_EXPERTS), lambda i: (i, 0)),
            pl.BlockSpec((N_EXPERTS, HIDDEN), lambda i: (0, 0)),
        ],
        out_specs=pl.BlockSpec((TT, HIDDEN), lambda i: (i, 0)),
        out_shape=jax.ShapeDtypeStruct((T, HIDDEN), jnp.float32),
        scratch_shapes=[pltpu.VMEM((N_EXPERTS, HIDDEN), jnp.bfloat16)],
    )(s, down)


def kernel(hidden_states, W_up, W_down, W_q, keys, up_embed, down_embed):
    x = hidden_states.reshape(T, HIDDEN)
    wq_b = W_q.astype(jnp.bfloat16)
    kk = keys.astype(jnp.bfloat16).transpose(2, 0, 1, 3).reshape(
        8, NUM_KEYS, DHALF)
    kbdt_b = jax.scipy.linalg.block_diag(*[kk[i].T for i in range(8)])

    h = _stage_h(x, W_up, W_down)
    h2, idxt, pwt = _stage_route(h, wq_b, kbdt_b, up_embed)
    s = _sc_combine(h2, idxt, pwt)
    out = _stage_out(s, down_embed)
    return out.reshape(1, T, HIDDEN)


# speedup vs baseline: 1.1087x; 1.0063x over previous
import functools

import jax
import jax.numpy as jnp
from jax import lax
from jax.experimental import pallas as pl
from jax.experimental.pallas import tpu as pltpu
from jax.experimental.pallas import tpu_sc as plsc

HIDDEN = 1024
SHARED = 4096
PRIVATE = 1024
N_EXPERTS = 4096
N_HEADS = 4
K_PER_HEAD = 4
NUM_KEYS = 64
DHALF = PRIVATE // 2
T = 2048

TT = 256
NEG = float("-inf")


def _h_body(x_ref, wup_ref, wdn_ref, h_ref, wupb_ref, wdnb_ref):
    @pl.when(pl.program_id(0) == 0)
    def _cast():
        wupb_ref[...] = wup_ref[...].astype(jnp.bfloat16)
        wdnb_ref[...] = wdn_ref[...].astype(jnp.bfloat16)

    xb = x_ref[...].astype(jnp.bfloat16)
    mid = jnp.dot(xb, wupb_ref[...], preferred_element_type=jnp.float32)
    midb = jax.nn.silu(mid).astype(jnp.bfloat16)
    h_ref[...] = jnp.dot(midb, wdnb_ref[...], preferred_element_type=jnp.float32)


def _stage_h(x, wup, wdn):
    return pl.pallas_call(
        _h_body,
        grid=(T // TT,),
        in_specs=[
            pl.BlockSpec((TT, HIDDEN), lambda i: (i, 0)),
            pl.BlockSpec((HIDDEN, SHARED), lambda i: (0, 0)),
            pl.BlockSpec((SHARED, PRIVATE), lambda i: (0, 0)),
        ],
        out_specs=pl.BlockSpec((TT, PRIVATE), lambda i: (i, 0)),
        out_shape=jax.ShapeDtypeStruct((T, PRIVATE), jnp.float32),
        scratch_shapes=[pltpu.VMEM((HIDDEN, SHARED), jnp.bfloat16),
                        pltpu.VMEM((SHARED, PRIVATE), jnp.bfloat16)],
    )(x, wup, wdn)


def _top4_t(s):
    n = s.shape[0]
    iota = lax.broadcasted_iota(jnp.int32, s.shape, 0)
    vals, poss = [], []
    for _ in range(K_PER_HEAD):
        m = jnp.max(s, axis=0, keepdims=True)
        hit = s == m
        pos = jnp.min(jnp.where(hit, iota, n), axis=0, keepdims=True)
        vals.append(m)
        poss.append(pos)
        s = jnp.where(iota == pos, NEG, s)
    return vals, poss


def _route_body(h_ref, wq_ref, kbdt_ref, up_ref, h2_ref, idx_ref, pw_ref,
                upb_ref):
    @pl.when(pl.program_id(0) == 0)
    def _cast():
        upb_ref[...] = up_ref[...].astype(jnp.bfloat16)

    hb = h_ref[...].astype(jnp.bfloat16)
    q = jnp.dot(hb, wq_ref[...], preferred_element_type=jnp.float32)
    h2_ref[...] = lax.dot_general(
        hb, upb_ref[...], (((1,), (1,)), ((), ())),
        preferred_element_type=jnp.float32)
    qb = q.astype(jnp.bfloat16)
    sim = jnp.dot(qb, kbdt_ref[...], preferred_element_type=jnp.float32)
    simt = sim.T

    idx_rows, pw_rows = [], []
    for hh in range(N_HEADS):
        rx = hh * NUM_KEYS
        ry = (N_HEADS + hh) * NUM_KEYS
        vx, ix = _top4_t(simt[rx:rx + NUM_KEYS, :])
        vy, iy = _top4_t(simt[ry:ry + NUM_KEYS, :])
        all_s = jnp.concatenate(
            [vx[i] + vy[j] for i in range(4) for j in range(4)], axis=0)
        all_i = jnp.concatenate(
            [ix[i] * NUM_KEYS + iy[j] for i in range(4) for j in range(4)], axis=0)
        iota16 = lax.broadcasted_iota(jnp.int32, all_s.shape, 0)
        s = all_s
        svals, eidx = [], []
        for _ in range(K_PER_HEAD):
            m = jnp.max(s, axis=0, keepdims=True)
            hit = s == m
            pos = jnp.min(jnp.where(hit, iota16, 16), axis=0, keepdims=True)
            e = jnp.sum(jnp.where(iota16 == pos, all_i, 0), axis=0, keepdims=True)
            svals.append(m)
            eidx.append(e)
            s = jnp.where(iota16 == pos, NEG, s)
        sc = jnp.concatenate(svals, axis=0)
        mx = jnp.max(sc, axis=0, keepdims=True)
        ex = jnp.exp(sc - mx)
        pw = ex / jnp.sum(ex, axis=0, keepdims=True)
        idx_rows.extend(eidx)
        pw_rows.append(pw)
    idx_ref[...] = jnp.concatenate(idx_rows, axis=0)
    pw_ref[...] = jnp.concatenate(pw_rows, axis=0)


def _stage_route(h, wq_b, kbdt_b, up):
    return pl.pallas_call(
        _route_body,
        grid=(T // TT,),
        in_specs=[
            pl.BlockSpec((TT, PRIVATE), lambda i: (i, 0)),
            pl.BlockSpec((PRIVATE, 2 * N_HEADS * DHALF), lambda i: (0, 0)),
            pl.BlockSpec((2 * N_HEADS * DHALF, 2 * N_HEADS * NUM_KEYS),
                         lambda i: (0, 0)),
            pl.BlockSpec((N_EXPERTS, PRIVATE), lambda i: (0, 0)),
        ],
        out_specs=[
            pl.BlockSpec((TT, N_EXPERTS), lambda i: (i, 0)),
            pl.BlockSpec((16, TT), lambda i: (0, i)),
            pl.BlockSpec((16, TT), lambda i: (0, i)),
        ],
        out_shape=[
            jax.ShapeDtypeStruct((T, N_EXPERTS), jnp.float32),
            jax.ShapeDtypeStruct((16, T), jnp.int32),
            jax.ShapeDtypeStruct((16, T), jnp.float32),
        ],
        scratch_shapes=[pltpu.VMEM((N_EXPERTS, PRIVATE), jnp.bfloat16)],
    )(h, wq_b, kbdt_b, up)


_NC, _NS = 2, 16
_NW = _NC * _NS
_TPW = T // _NW
_TBLK = 4
_NBLK = _TPW // _TBLK


def _sc_combine(h2, idxt, pwt):
    mesh = plsc.VectorSubcoreMesh(core_axis_name="c", subcore_axis_name="s")

    @functools.partial(
        pl.kernel,
        mesh=mesh,
        out_type=jax.ShapeDtypeStruct((T, N_EXPERTS), jnp.float32),
        compiler_params=pltpu.CompilerParams(needs_layout_passes=False),
        scratch_types=[
            pltpu.VMEM((16, 128), jnp.int32),
            pltpu.VMEM((16, 128), jnp.float32),
            pltpu.VMEM((2, _TBLK, N_EXPERTS), jnp.float32),
            pltpu.VMEM((2, _TBLK, N_EXPERTS), jnp.float32),
            pltpu.SemaphoreType.DMA,
            pltpu.SemaphoreType.DMA,
            pltpu.SemaphoreType.DMA,
            pltpu.SemaphoreType.DMA,
        ],
    )
    def sck(h2_hbm, idx_hbm, pw_hbm, s_hbm, idx_v, pw_v, hbuf, sbuf,
            isem0, isem1, osem0, osem1):
        wid = lax.axis_index("s") * _NC + lax.axis_index("c")
        base = wid * _TPW
        slab = (wid // 2) * 128
        off = (wid % 2) * _TPW
        isems = [isem0, isem1]
        osems = [osem0, osem1]
        pltpu.sync_copy(idx_hbm.at[:, pl.ds(slab, 128)], idx_v)
        pltpu.sync_copy(pw_hbm.at[:, pl.ds(slab, 128)], pw_v)

        for k0 in range(2):
            for r0 in range(_TBLK):
                def zero_body(ci, carry, k0=k0, r0=r0):
                    for j in range(16):
                        sbuf[k0, r0, pl.ds(ci * 256 + j * 16, 16)] = (
                            jnp.zeros((16,), jnp.float32))
                    return carry

                lax.fori_loop(0, N_EXPERTS // 256, zero_body, 0)

        def in_copy(bb, k):
            return pltpu.make_async_copy(
                h2_hbm.at[pl.ds(base + bb * _TBLK, _TBLK)], hbuf.at[k],
                isems[k])

        def out_copy(bb, k):
            return pltpu.make_async_copy(
                sbuf.at[k], s_hbm.at[pl.ds(base + bb * _TBLK, _TBLK)],
                osems[k])

        lane = lax.iota(jnp.int32, 16)
        in_copy(0, 0).start()
        for bb in range(_NBLK):
            k = bb % 2
            if bb + 1 < _NBLK:
                in_copy(bb + 1, 1 - k).start()
            in_copy(bb, k).wait()
            if bb >= 2:
                out_copy(bb - 2, k).wait()
                for i in range(_TBLK):
                    tl = (bb - 2) * _TBLK + i
                    rowi = jnp.full((16,), i, jnp.int32)
                    e16 = plsc.load_gather(
                        idx_v, [lane, jnp.full((16,), off + tl, jnp.int32)])
                    plsc.store_scatter(sbuf, [jnp.full((16,), k, jnp.int32),
                                              rowi, e16],
                                       jnp.zeros((16,), jnp.float32))
            for i in range(_TBLK):
                tl = bb * _TBLK + i
                rowi = jnp.full((16,), i, jnp.int32)
                k16 = jnp.full((16,), k, jnp.int32)
                e16 = plsc.load_gather(
                    idx_v, [lane, jnp.full((16,), off + tl, jnp.int32)])
                x16 = plsc.load_gather(hbuf, [k16, rowi, e16])
                pw16 = plsc.load_gather(
                    pw_v, [lane, jnp.full((16,), off + tl, jnp.int32)])
                w = x16 * pw16 / (1.0 + jnp.exp(-x16))
                for hh in range(N_HEADS):
                    plsc.addupdate_scatter(
                        sbuf, [k16, rowi, e16], w, mask=(lane // 4) == hh)
            out_copy(bb, k).start()
        out_copy(_NBLK - 2, _NBLK % 2).wait()
        out_copy(_NBLK - 1, (_NBLK - 1) % 2).wait()

    return sck(h2, idxt, pwt)


def _out_body(s_ref, down_ref, o_ref):
    sb = s_ref[...].astype(jnp.bfloat16)
    o_ref[...] = jnp.dot(sb, down_ref[...], preferred_element_type=jnp.float32)


def _stage_out(s, down):
    return pl.pallas_call(
        _out_body,
        grid=(T // TT,),
        in_specs=[
            pl.BlockSpec((TT, N_EXPERTS), lambda i: (i, 0)),
            pl.BlockSpec((N_EXPERTS, HIDDEN), lambda i: (0, 0)),
        ],
        out_specs=pl.BlockSpec((TT, HIDDEN), lambda i: (i, 0)),
        out_shape=jax.ShapeDtypeStruct((T, HIDDEN), jnp.float32),
    )(s, down)


def kernel(hidden_states, W_up, W_down, W_q, keys, up_embed, down_embed):
    x = hidden_states.reshape(T, HIDDEN)
    wq_b = W_q.astype(jnp.bfloat16)
    down_b = down_embed.astype(jnp.bfloat16)
    kk = keys.astype(jnp.bfloat16).transpose(2, 0, 1, 3).reshape(
        8, NUM_KEYS, DHALF)
    kbdt_b = jax.scipy.linalg.block_diag(*[kk[i].T for i in range(8)])

    h = _stage_h(x, W_up, W_down)
    h2, idxt, pwt = _stage_route(h, wq_b, kbdt_b, up_embed)
    s = _sc_combine(h2, idxt, pwt)
    out = _stage_out(s, down_b)
    return out.reshape(1, T, HIDDEN)


# speedup vs baseline: 1.1254x; 1.0151x over previous
import functools

import jax
import jax.numpy as jnp
from jax import lax
from jax.experimental import pallas as pl
from jax.experimental.pallas import tpu as pltpu
from jax.experimental.pallas import tpu_sc as plsc

HIDDEN = 1024
SHARED = 4096
PRIVATE = 1024
N_EXPERTS = 4096
N_HEADS = 4
K_PER_HEAD = 4
NUM_KEYS = 64
DHALF = PRIVATE // 2
T = 2048

TT = 256
NEG = float("-inf")


def _h_body(x_ref, wup_ref, wdn_ref, h_ref, wupb_ref, wdnb_ref):
    @pl.when(pl.program_id(0) == 0)
    def _cast():
        wupb_ref[...] = wup_ref[...].astype(jnp.bfloat16)
        wdnb_ref[...] = wdn_ref[...].astype(jnp.bfloat16)

    xb = x_ref[...].astype(jnp.bfloat16)
    mid = jnp.dot(xb, wupb_ref[...], preferred_element_type=jnp.float32)
    midb = jax.nn.silu(mid).astype(jnp.bfloat16)
    h_ref[...] = jnp.dot(midb, wdnb_ref[...], preferred_element_type=jnp.float32)


def _stage_h(x, wup, wdn):
    return pl.pallas_call(
        _h_body,
        grid=(T // TT,),
        in_specs=[
            pl.BlockSpec((TT, HIDDEN), lambda i: (i, 0)),
            pl.BlockSpec((HIDDEN, SHARED), lambda i: (0, 0)),
            pl.BlockSpec((SHARED, PRIVATE), lambda i: (0, 0)),
        ],
        out_specs=pl.BlockSpec((TT, PRIVATE), lambda i: (i, 0)),
        out_shape=jax.ShapeDtypeStruct((T, PRIVATE), jnp.float32),
        scratch_shapes=[pltpu.VMEM((HIDDEN, SHARED), jnp.bfloat16),
                        pltpu.VMEM((SHARED, PRIVATE), jnp.bfloat16)],
    )(x, wup, wdn)


def _top4_t(s):
    n = s.shape[0]
    iota = lax.broadcasted_iota(jnp.int32, s.shape, 0)
    vals, poss = [], []
    for _ in range(K_PER_HEAD):
        m = jnp.max(s, axis=0, keepdims=True)
        hit = s == m
        pos = jnp.min(jnp.where(hit, iota, n), axis=0, keepdims=True)
        vals.append(m)
        poss.append(pos)
        s = jnp.where(iota == pos, NEG, s)
    return vals, poss


def _route_body(h_ref, wq_ref, kbdt_ref, up_ref, h2_ref, idx_ref, pw_ref,
                upb_ref):
    @pl.when(pl.program_id(0) == 0)
    def _cast():
        upb_ref[...] = up_ref[...].astype(jnp.bfloat16)

    hb = h_ref[...].astype(jnp.bfloat16)
    q = jnp.dot(hb, wq_ref[...], preferred_element_type=jnp.float32)
    h2 = lax.dot_general(
        hb, upb_ref[...], (((1,), (1,)), ((), ())),
        preferred_element_type=jnp.float32)
    h2_ref[...] = h2.reshape(TT * N_EXPERTS)
    qb = q.astype(jnp.bfloat16)
    sim = jnp.dot(qb, kbdt_ref[...], preferred_element_type=jnp.float32)
    simt = sim.T

    idx_rows, pw_rows = [], []
    for hh in range(N_HEADS):
        rx = hh * NUM_KEYS
        ry = (N_HEADS + hh) * NUM_KEYS
        vx, ix = _top4_t(simt[rx:rx + NUM_KEYS, :])
        vy, iy = _top4_t(simt[ry:ry + NUM_KEYS, :])
        all_s = jnp.concatenate(
            [vx[i] + vy[j] for i in range(4) for j in range(4)], axis=0)
        all_i = jnp.concatenate(
            [ix[i] * NUM_KEYS + iy[j] for i in range(4) for j in range(4)], axis=0)
        iota16 = lax.broadcasted_iota(jnp.int32, all_s.shape, 0)
        s = all_s
        svals, eidx = [], []
        for _ in range(K_PER_HEAD):
            m = jnp.max(s, axis=0, keepdims=True)
            hit = s == m
            pos = jnp.min(jnp.where(hit, iota16, 16), axis=0, keepdims=True)
            e = jnp.sum(jnp.where(iota16 == pos, all_i, 0), axis=0, keepdims=True)
            svals.append(m)
            eidx.append(e)
            s = jnp.where(iota16 == pos, NEG, s)
        sc = jnp.concatenate(svals, axis=0)
        mx = jnp.max(sc, axis=0, keepdims=True)
        ex = jnp.exp(sc - mx)
        pw = ex / jnp.sum(ex, axis=0, keepdims=True)
        idx_rows.extend(eidx)
        pw_rows.append(pw)
    idx_ref[...] = jnp.concatenate(idx_rows, axis=0)
    pw_ref[...] = jnp.concatenate(pw_rows, axis=0)


def _stage_route(h, wq_b, kbdt_b, up):
    return pl.pallas_call(
        _route_body,
        grid=(T // TT,),
        in_specs=[
            pl.BlockSpec((TT, PRIVATE), lambda i: (i, 0)),
            pl.BlockSpec((PRIVATE, 2 * N_HEADS * DHALF), lambda i: (0, 0)),
            pl.BlockSpec((2 * N_HEADS * DHALF, 2 * N_HEADS * NUM_KEYS),
                         lambda i: (0, 0)),
            pl.BlockSpec((N_EXPERTS, PRIVATE), lambda i: (0, 0)),
        ],
        out_specs=[
            pl.BlockSpec((TT * N_EXPERTS,), lambda i: (i,)),
            pl.BlockSpec((16, TT), lambda i: (0, i)),
            pl.BlockSpec((16, TT), lambda i: (0, i)),
        ],
        out_shape=[
            jax.ShapeDtypeStruct((T * N_EXPERTS,), jnp.float32),
            jax.ShapeDtypeStruct((16, T), jnp.int32),
            jax.ShapeDtypeStruct((16, T), jnp.float32),
        ],
        scratch_shapes=[pltpu.VMEM((N_EXPERTS, PRIVATE), jnp.bfloat16)],
    )(h, wq_b, kbdt_b, up)


_NC, _NS = 2, 16
_NW = _NC * _NS
_TPW = T // _NW
_TBLK = 4
_NBLK = _TPW // _TBLK


def _sc_combine(h2, idxt, pwt):
    mesh = plsc.VectorSubcoreMesh(core_axis_name="c", subcore_axis_name="s")

    @functools.partial(
        pl.kernel,
        mesh=mesh,
        out_type=jax.ShapeDtypeStruct((T, N_EXPERTS), jnp.float32),
        compiler_params=pltpu.CompilerParams(needs_layout_passes=False),
        scratch_types=[
            pltpu.VMEM((16, 128), jnp.int32),
            pltpu.VMEM((16, 128), jnp.float32),
            pltpu.VMEM((_TPW * 16 // 128, 128), jnp.int32),
            pltpu.VMEM((_TPW * 16,), jnp.float32),
            pltpu.VMEM((2, _TBLK, N_EXPERTS), jnp.float32),
            pltpu.SemaphoreType.DMA,
            pltpu.SemaphoreType.DMA,
            pltpu.SemaphoreType.DMA,
        ],
    )
    def sck(h2_hbm, idx_hbm, pw_hbm, s_hbm, idx_v, pw_v, fidx, xall, sbuf,
            gsem, osem0, osem1):
        wid = lax.axis_index("s") * _NC + lax.axis_index("c")
        base = wid * _TPW
        slab = (wid // 2) * 128
        off = (wid % 2) * _TPW
        osems = [osem0, osem1]
        pltpu.sync_copy(idx_hbm.at[:, pl.ds(slab, 128)], idx_v)
        pltpu.sync_copy(pw_hbm.at[:, pl.ds(slab, 128)], pw_v)

        def fidx_body(tl, carry):
            lane_ = lax.iota(jnp.int32, 16)
            e16 = plsc.load_gather(
                idx_v, [lane_, jnp.full((16,), off + tl, jnp.int32)])
            flat = (base + tl) * N_EXPERTS + e16
            fidx[tl // 8, pl.ds((tl % 8) * 16, 16)] = flat
            return carry

        lax.fori_loop(0, _TPW, fidx_body, 0)
        for c in range(_TPW * 16 // 128):
            pltpu.make_async_copy(h2_hbm.at[fidx.at[c]],
                                  xall.at[pl.ds(c * 128, 128)], gsem).start()
        for c in range(_TPW * 16 // 128):
            pltpu.make_async_copy(h2_hbm.at[fidx.at[c]],
                                  xall.at[pl.ds(c * 128, 128)], gsem).wait()

        for k0 in range(2):
            for r0 in range(_TBLK):
                def zero_body(ci, carry, k0=k0, r0=r0):
                    for j in range(16):
                        sbuf[k0, r0, pl.ds(ci * 256 + j * 16, 16)] = (
                            jnp.zeros((16,), jnp.float32))
                    return carry

                lax.fori_loop(0, N_EXPERTS // 256, zero_body, 0)

        def out_copy(bb, k):
            return pltpu.make_async_copy(
                sbuf.at[k], s_hbm.at[pl.ds(base + bb * _TBLK, _TBLK)],
                osems[k])

        lane = lax.iota(jnp.int32, 16)
        for bb in range(_NBLK):
            k = bb % 2
            if bb >= 2:
                out_copy(bb - 2, k).wait()
                for i in range(_TBLK):
                    tl = (bb - 2) * _TBLK + i
                    rowi = jnp.full((16,), i, jnp.int32)
                    e16 = plsc.load_gather(
                        idx_v, [lane, jnp.full((16,), off + tl, jnp.int32)])
                    plsc.store_scatter(sbuf, [jnp.full((16,), k, jnp.int32),
                                              rowi, e16],
                                       jnp.zeros((16,), jnp.float32))
            for i in range(_TBLK):
                tl = bb * _TBLK + i
                rowi = jnp.full((16,), i, jnp.int32)
                k16 = jnp.full((16,), k, jnp.int32)
                e16 = plsc.load_gather(
                    idx_v, [lane, jnp.full((16,), off + tl, jnp.int32)])
                x16 = xall[pl.ds(tl * 16, 16)]
                pw16 = plsc.load_gather(
                    pw_v, [lane, jnp.full((16,), off + tl, jnp.int32)])
                w = x16 * pw16 / (1.0 + jnp.exp(-x16))
                for hh in range(N_HEADS):
                    plsc.addupdate_scatter(
                        sbuf, [k16, rowi, e16], w, mask=(lane // 4) == hh)
            out_copy(bb, k).start()
        out_copy(_NBLK - 2, _NBLK % 2).wait()
        out_copy(_NBLK - 1, (_NBLK - 1) % 2).wait()

    return sck(h2, idxt, pwt)


def _out_body(s_ref, down_ref, o_ref):
    sb = s_ref[...].astype(jnp.bfloat16)
    o_ref[...] = jnp.dot(sb, down_ref[...], preferred_element_type=jnp.float32)


def _stage_out(s, down):
    return pl.pallas_call(
        _out_body,
        grid=(T // TT,),
        in_specs=[
            pl.BlockSpec((TT, N_EXPERTS), lambda i: (i, 0)),
            pl.BlockSpec((N_EXPERTS, HIDDEN), lambda i: (0, 0)),
        ],
        out_specs=pl.BlockSpec((TT, HIDDEN), lambda i: (i, 0)),
        out_shape=jax.ShapeDtypeStruct((T, HIDDEN), jnp.float32),
    )(s, down)


def kernel(hidden_states, W_up, W_down, W_q, keys, up_embed, down_embed):
    x = hidden_states.reshape(T, HIDDEN)
    wq_b = W_q.astype(jnp.bfloat16)
    down_b = down_embed.astype(jnp.bfloat16)
    kk = keys.astype(jnp.bfloat16).transpose(2, 0, 1, 3).reshape(
        8, NUM_KEYS, DHALF)
    kbdt_b = jax.scipy.linalg.block_diag(*[kk[i].T for i in range(8)])

    h = _stage_h(x, W_up, W_down)
    h2, idxt, pwt = _stage_route(h, wq_b, kbdt_b, up_embed)
    s = _sc_combine(h2, idxt, pwt)
    out = _stage_out(s, down_b)
    return out.reshape(1, T, HIDDEN)
